# baseline (speedup 1.0000x reference)
import jax
import jax.numpy as jnp
import numpy as np
from jax import lax
from jax.experimental import pallas as pl
from jax.experimental.pallas import tpu as pltpu
from jax.experimental.pallas import tpu_sc as plsc

VOCAB = 100000
Q_HEADS = 16
KV_HEADS = 4
HEAD_DIM = 64
B = 32
CTX = 4096

C = 64
NCHUNK = CTX // C
NBUF = 3
VS = 28
CTXSPLIT = VS * C
D_KV = KV_HEADS * HEAD_DIM
D_Q = Q_HEADS * HEAD_DIM
NQUART = HEAD_DIM // 16


def _rope_tables():
    pos = np.arange(CTX + 1, dtype=np.float64)
    inv_freq = 1.0 / 10000.0 ** (
        np.arange(0, HEAD_DIM, 2, dtype=np.float64) / HEAD_DIM)
    freqs = pos[:, None] * inv_freq[None, :]
    emb = np.repeat(freqs, 2, axis=-1)
    cos = np.cos(emb).astype(np.float32)
    sign = np.where(np.arange(HEAD_DIM) % 2 == 0, -1.0, 1.0)
    sin_s = (np.sin(emb) * sign[None, :]).astype(np.float32)
    return cos, sin_s


def _body(ctx_hbm, nxt_hbm, wq_hbm, wk_hbm, wv_hbm, cs_hbm, csq_hbm,
          q_hbm, k_hbm, v_hbm,
          idx_v, kbuf, vbuf, csbuf, qidx1, qbuf, qout, csqb,
          gsem0, gsem1, gsem2, ssem0, ssem1, ssem2):
    nc = 2
    b = lax.axis_index("s") * nc + lax.axis_index("c")
    gsem = (gsem0, gsem1, gsem2)
    ssem = (ssem0, ssem1, ssem2)
    base_h = b * KV_HEADS

    lane = lax.iota(jnp.int32, 16)
    perm_col = lane ^ 1
    zero16 = lane * 0

    def start_gather(i, nb, with_v):
        pltpu.async_copy(wk_hbm.at[idx_v.at[i]], kbuf.at[nb], gsem[nb])
        if with_v:
            pltpu.async_copy(wv_hbm.at[idx_v.at[i]], vbuf.at[nb], gsem[nb])
        pltpu.async_copy(cs_hbm.at[i], csbuf.at[nb], gsem[nb])

    def drain_gather(nb, with_v):
        pltpu.make_async_copy(wk_hbm.at[pl.ds(0, C)], kbuf.at[nb],
                              gsem[nb]).wait()
        if with_v:
            pltpu.make_async_copy(wv_hbm.at[pl.ds(0, C)], vbuf.at[nb],
                                  gsem[nb]).wait()
        pltpu.make_async_copy(cs_hbm.at[0], csbuf.at[nb], gsem[nb]).wait()

    def start_scatter(i, nb, with_v):
        for h in range(KV_HEADS):
            pltpu.async_copy(kbuf.at[nb, :, pl.ds(h * HEAD_DIM, HEAD_DIM)],
                             k_hbm.at[base_h + h, pl.ds(i * C, C)], ssem[nb])
            if with_v:
                pltpu.async_copy(
                    vbuf.at[nb, :, pl.ds(h * HEAD_DIM, HEAD_DIM)],
                    v_hbm.at[base_h + h, pl.ds(i * C - CTXSPLIT, C)],
                    ssem[nb])

    def drain_scatter(nb, with_v):
        n = (2 if with_v else 1) * KV_HEADS
        for _ in range(n):
            pltpu.make_async_copy(
                k_hbm.at[0, pl.ds(0, C)],
                kbuf.at[nb, :, pl.ds(0, HEAD_DIM)], ssem[nb]).wait()

    def rope(nb):
        def rope_t(t, carry):
            for quart in range(NQUART):
                c = csbuf[nb, t, pl.ds(quart * 16, 16)]
                s = csbuf[nb, t, pl.ds(HEAD_DIM + quart * 16, 16)]
                for h in range(KV_HEADS):
                    off = h * HEAD_DIM + quart * 16
                    x = kbuf[nb, t, pl.ds(off, 16)]
                    xs = plsc.load_gather(
                        kbuf.at[nb], [zero16 + t, perm_col + off])
                    kbuf[nb, t, pl.ds(off, 16)] = x * c + xs * s
            return carry
        lax.fori_loop(0, C, rope_t, 0)

    def body(i, nb, prefetch, drain_prev, with_v):
        drain_gather(nb, with_v)
        rope(nb)
        start_scatter(i, nb, with_v)
        pb = (nb + 2) % NBUF
        if drain_prev:
            drain_scatter(pb, with_v)
        if prefetch:
            start_gather(i + 2, pb, with_v)

    def emit_phase(s0, L, with_v):
        start_gather(s0, 0, with_v)
        start_gather(s0 + 1, 1, with_v)
        body(s0, 0, True, False, with_v)
        if L % 3 == 1:
            m = (L - 4) // 3
        else:
            m = (L - 3) // 3

        def triple(g, carry):
            i = s0 + 3 * g + 1
            body(i, 1, True, True, with_v)
            body(i + 1, 2, True, True, with_v)
            body(i + 2, 0, True, True, with_v)
            return carry

        lax.fori_loop(0, m, triple, 0)
        if L % 3 == 1:
            body(s0 + L - 3, (L - 3) % 3, True, True, with_v)
        body(s0 + L - 2, (L - 2) % 3, False, True, with_v)
        body(s0 + L - 1, (L - 1) % 3, False, True, with_v)
        drain_scatter((L - 1) % 3, with_v)

    pltpu.sync_copy(ctx_hbm.at[b], idx_v)
    pltpu.sync_copy(nxt_hbm.at[b, pl.ds(0, 1)], qidx1)
    pltpu.async_copy(wq_hbm.at[qidx1], qbuf, gsem2)
    pltpu.make_async_copy(wq_hbm.at[pl.ds(0, 1)], qbuf, gsem2).wait()
    pltpu.sync_copy(csq_hbm, csqb)
    for j in range(D_Q // 16):
        quart = j % NQUART
        c = csqb[pl.ds(quart * 16, 16)]
        s = csqb[pl.ds(HEAD_DIM + quart * 16, 16)]
        x = qbuf[0, pl.ds(j * 16, 16)]
        xs = plsc.load_gather(qbuf, [zero16, perm_col + j * 16])
        qout[pl.ds(j * 16, 16)] = x * c + xs * s
    pltpu.sync_copy(qout, q_hbm.at[b])

    emit_phase(0, VS, False)
    emit_phase(VS, NCHUNK - VS, True)


def _tc_v_body(tok_hbm, wv_hbm, out_ref, idx_s, vbuf, sem_i, sem_g):
    b = pl.program_id(0)
    cp = pltpu.make_async_copy(tok_hbm.at[b, pl.ds(0, CTXSPLIT)], idx_s,
                               sem_i)
    cp.start()
    cp.wait()

    def issue(j, carry):
        pltpu.make_async_copy(wv_hbm.at[idx_s[j]], vbuf.at[j], sem_g).start()
        return carry

    lax.fori_loop(0, CTXSPLIT, issue, 0)

    def waitone(j, carry):
        pltpu.make_async_copy(wv_hbm.at[0], vbuf.at[0], sem_g).wait()
        return carry

    lax.fori_loop(0, CTXSPLIT, waitone, 0)
    for h in range(KV_HEADS):
        out_ref[0, h] = vbuf[:, h * HEAD_DIM:(h + 1) * HEAD_DIM]


@jax.jit
def _call(ctx_tok, ctx3, nxt8, Wq, Wk, Wv):
    cos, sin_s = _rope_tables()
    cs_k = np.concatenate(
        [cos[:CTX].reshape(NCHUNK, C, HEAD_DIM),
         sin_s[:CTX].reshape(NCHUNK, C, HEAD_DIM)], axis=-1)
    csq = np.concatenate([cos[CTX], sin_s[CTX]])
    mesh = plsc.VectorSubcoreMesh(core_axis_name="c", subcore_axis_name="s")
    f = pl.kernel(
        _body,
        out_type=[
            jax.ShapeDtypeStruct((B, D_Q), jnp.float32),
            jax.ShapeDtypeStruct((B * KV_HEADS, CTX, HEAD_DIM), jnp.float32),
            jax.ShapeDtypeStruct((B * KV_HEADS, CTX - CTXSPLIT, HEAD_DIM),
                                 jnp.float32),
        ],
        mesh=mesh,
        compiler_params=pltpu.CompilerParams(use_tc_tiling_on_sc=False,
                                             needs_layout_passes=False),
        scratch_types=[
            pltpu.VMEM((NCHUNK, C), jnp.int32),
            pltpu.VMEM((NBUF, C, D_KV), jnp.float32),
            pltpu.VMEM((NBUF, C, D_KV), jnp.float32),
            pltpu.VMEM((NBUF, C, 2 * HEAD_DIM), jnp.float32),
            pltpu.VMEM((1,), jnp.int32),
            pltpu.VMEM((1, D_Q), jnp.float32),
            pltpu.VMEM((D_Q,), jnp.float32),
            pltpu.VMEM((2 * HEAD_DIM,), jnp.float32),
            pltpu.SemaphoreType.DMA,
            pltpu.SemaphoreType.DMA,
            pltpu.SemaphoreType.DMA,
            pltpu.SemaphoreType.DMA,
            pltpu.SemaphoreType.DMA,
            pltpu.SemaphoreType.DMA,
        ],
    )
    q, k, v_sc = f(ctx3, nxt8, Wq, Wk, Wv, jnp.asarray(cs_k),
                   jnp.asarray(csq))

    v_tc = pl.pallas_call(
        _tc_v_body,
        grid=(B,),
        in_specs=[
            pl.BlockSpec(memory_space=pl.ANY),
            pl.BlockSpec(memory_space=pl.ANY),
        ],
        out_specs=pl.BlockSpec(
            (1, KV_HEADS, CTXSPLIT, HEAD_DIM),
            lambda b: (b, 0, 0, 0)),
        out_shape=jax.ShapeDtypeStruct((B, KV_HEADS, CTXSPLIT, HEAD_DIM),
                                       jnp.float32),
        scratch_shapes=[
            pltpu.SMEM((CTXSPLIT,), jnp.int32),
            pltpu.VMEM((CTXSPLIT, D_KV), jnp.float32),
            pltpu.SemaphoreType.DMA,
            pltpu.SemaphoreType.DMA,
        ],
    )(ctx_tok, Wv)

    q = q.reshape(B, Q_HEADS, 1, HEAD_DIM)
    k = k.reshape(B, KV_HEADS, CTX, HEAD_DIM)
    v_sc = v_sc.reshape(B, KV_HEADS, CTX - CTXSPLIT, HEAD_DIM)
    v = jnp.concatenate([v_tc, v_sc], axis=2)
    return q, k, v


def kernel(context_tokens, next_tokens, Wq, Wk, Wv):
    ctx3 = context_tokens.reshape(B, NCHUNK, C)
    nxt8 = jnp.broadcast_to(next_tokens[:, None], (B, 8))
    return _call(context_tokens, ctx3, nxt8, Wq, Wk, Wv)


# speedup vs baseline: 1.4854x; 1.4854x over previous
import jax
import jax.numpy as jnp
import numpy as np
from jax import lax
from jax.experimental import pallas as pl
from jax.experimental.pallas import tpu as pltpu
from jax.experimental.pallas import tpu_sc as plsc

VOCAB = 100000
Q_HEADS = 16
KV_HEADS = 4
HEAD_DIM = 64
B = 32
CTX = 4096

C = 64
NCHUNK = CTX // C
NBUF = 3
D_KV = KV_HEADS * HEAD_DIM
D_Q = Q_HEADS * HEAD_DIM
NQUART = HEAD_DIM // 16


def _rope_tables():
    pos = np.arange(CTX + 1, dtype=np.float64)
    inv_freq = 1.0 / 10000.0 ** (
        np.arange(0, HEAD_DIM, 2, dtype=np.float64) / HEAD_DIM)
    freqs = pos[:, None] * inv_freq[None, :]
    emb = np.repeat(freqs, 2, axis=-1)
    cos = np.cos(emb).astype(np.float32)
    sign = np.where(np.arange(HEAD_DIM) % 2 == 0, -1.0, 1.0)
    sin_s = (np.sin(emb) * sign[None, :]).astype(np.float32)
    return cos, sin_s


def _body(ctx_hbm, nxt_hbm, wq_hbm, wk_hbm, wv_hbm, cs_hbm, csq_hbm,
          q_hbm, k_hbm, v_hbm,
          idx_v, kbuf, vbuf, csbuf, qidx1, qbuf, qout, csqb,
          gsem0, gsem1, gsem2, ssem0, ssem1, ssem2):
    nc = 2
    b = lax.axis_index("s") * nc + lax.axis_index("c")
    gsem = (gsem0, gsem1, gsem2)
    ssem = (ssem0, ssem1, ssem2)
    base_h = b * KV_HEADS

    lane = lax.iota(jnp.int32, 16)
    perm_col = lane ^ 1
    zero16 = lane * 0

    def start_gather(i, nb):
        pltpu.async_copy(wk_hbm.at[idx_v.at[i]], kbuf.at[nb], gsem[nb])
        pltpu.async_copy(wv_hbm.at[idx_v.at[i]], vbuf.at[nb], gsem[nb])
        pltpu.async_copy(cs_hbm.at[i], csbuf.at[nb], gsem[nb])

    def drain_gather(nb):
        pltpu.make_async_copy(wk_hbm.at[pl.ds(0, C)], kbuf.at[nb],
                              gsem[nb]).wait()
        pltpu.make_async_copy(wv_hbm.at[pl.ds(0, C)], vbuf.at[nb],
                              gsem[nb]).wait()
        pltpu.make_async_copy(cs_hbm.at[0], csbuf.at[nb], gsem[nb]).wait()

    def start_scatter(i, nb):
        for h in range(KV_HEADS):
            pltpu.async_copy(kbuf.at[nb, :, pl.ds(h * HEAD_DIM, HEAD_DIM)],
                             k_hbm.at[base_h + h, pl.ds(i * C, C)], ssem[nb])
            pltpu.async_copy(vbuf.at[nb, :, pl.ds(h * HEAD_DIM, HEAD_DIM)],
                             v_hbm.at[base_h + h, pl.ds(i * C, C)], ssem[nb])

    def drain_scatter(nb):
        for _ in range(2 * KV_HEADS):
            pltpu.make_async_copy(
                k_hbm.at[0, pl.ds(0, C)],
                kbuf.at[nb, :, pl.ds(0, HEAD_DIM)], ssem[nb]).wait()

    def rope_chunk(nb):
        def tok(t, carry):
            for j in range(D_KV // 16):
                quart = j % NQUART
                c = csbuf[nb, t, pl.ds(quart * 16, 16)]
                s = csbuf[nb, t, pl.ds(HEAD_DIM + quart * 16, 16)]
                x = kbuf[nb, t, pl.ds(j * 16, 16)]
                xs = plsc.load_gather(
                    kbuf, [zero16 + nb, zero16 + t, perm_col + j * 16])
                kbuf[nb, t, pl.ds(j * 16, 16)] = x * c + xs * s
            return carry
        lax.fori_loop(0, C, tok, 0)

    def body(i, nb, prefetch, drain_prev):
        drain_gather(nb)
        rope_chunk(nb)
        start_scatter(i, nb)
        pb = (nb + 2) % NBUF
        if drain_prev:
            drain_scatter(pb)
        if prefetch:
            start_gather(i + 2, pb)

    pltpu.sync_copy(ctx_hbm.at[b], idx_v)
    start_gather(0, 0)
    start_gather(1, 1)

    pltpu.sync_copy(nxt_hbm.at[b, pl.ds(0, 1)], qidx1)
    pltpu.async_copy(wq_hbm.at[qidx1], qbuf, gsem2).wait()
    pltpu.sync_copy(csq_hbm, csqb)
    for j in range(D_Q // 16):
        quart = j % NQUART
        c = csqb[pl.ds(quart * 16, 16)]
        s = csqb[pl.ds(HEAD_DIM + quart * 16, 16)]
        x = qbuf[0, pl.ds(j * 16, 16)]
        xs = plsc.load_gather(qbuf, [zero16, perm_col + j * 16])
        qout[pl.ds(j * 16, 16)] = x * c + xs * s
    pltpu.sync_copy(qout, q_hbm.at[b])

    body(0, 0, True, False)

    def triple(g, carry):
        i = 3 * g + 1
        body(i, 1, True, True)
        body(i + 1, 2, True, True)
        body(i + 2, 0, True, True)
        return carry

    lax.fori_loop(0, (NCHUNK - 4) // 3, triple, 0)

    body(NCHUNK - 3, 1, True, True)
    body(NCHUNK - 2, 2, False, True)
    body(NCHUNK - 1, 0, False, True)
    drain_scatter(0)


@jax.jit
def _sc_call(ctx3, nxt8, Wq, Wk, Wv):
    cos, sin_s = _rope_tables()
    cs_k = np.concatenate(
        [cos[:CTX].reshape(NCHUNK, C, HEAD_DIM),
         sin_s[:CTX].reshape(NCHUNK, C, HEAD_DIM)], axis=-1)
    csq = np.concatenate([cos[CTX], sin_s[CTX]])
    mesh = plsc.VectorSubcoreMesh(core_axis_name="c", subcore_axis_name="s")
    f = pl.kernel(
        _body,
        out_type=[
            jax.ShapeDtypeStruct((B, D_Q), jnp.float32),
            jax.ShapeDtypeStruct((B * KV_HEADS, CTX, HEAD_DIM), jnp.float32),
            jax.ShapeDtypeStruct((B * KV_HEADS, CTX, HEAD_DIM), jnp.float32),
        ],
        mesh=mesh,
        compiler_params=pltpu.CompilerParams(use_tc_tiling_on_sc=False,
                                             needs_layout_passes=False),
        scratch_types=[
            pltpu.VMEM((NCHUNK, C), jnp.int32),
            pltpu.VMEM((NBUF, C, D_KV), jnp.float32),
            pltpu.VMEM((NBUF, C, D_KV), jnp.float32),
            pltpu.VMEM((NBUF, C, 2 * HEAD_DIM), jnp.float32),
            pltpu.VMEM((1,), jnp.int32),
            pltpu.VMEM((1, D_Q), jnp.float32),
            pltpu.VMEM((D_Q,), jnp.float32),
            pltpu.VMEM((2 * HEAD_DIM,), jnp.float32),
            pltpu.SemaphoreType.DMA,
            pltpu.SemaphoreType.DMA,
            pltpu.SemaphoreType.DMA,
            pltpu.SemaphoreType.DMA,
            pltpu.SemaphoreType.DMA,
            pltpu.SemaphoreType.DMA,
        ],
    )
    return f(ctx3, nxt8, Wq, Wk, Wv, jnp.asarray(cs_k), jnp.asarray(csq))


def kernel(context_tokens, next_tokens, Wq, Wk, Wv):
    ctx3 = context_tokens.reshape(B, NCHUNK, C)
    nxt8 = jnp.broadcast_to(next_tokens[:, None], (B, 8))
    q, k, v = _sc_call(ctx3, nxt8, Wq, Wk, Wv)
    q = q.reshape(B, Q_HEADS, 1, HEAD_DIM)
    k = k.reshape(B, KV_HEADS, CTX, HEAD_DIM)
    v = v.reshape(B, KV_HEADS, CTX, HEAD_DIM)
    return q, k, v
